# baseline (speedup 1.0000x reference)
import functools

import jax
import jax.numpy as jnp
from jax import lax
from jax.experimental import pallas as pl
from jax.experimental.pallas import tpu as pltpu
from jax.experimental.pallas import tpu_sc as plsc

_CLS_NUM_LIST = [5000 // (i + 1) for i in range(100)]
_MAX_M = 0.5
_S = 30.0

_B = 16384
_C = 100
_NW = 32
_CPW = _B // _NW
_NCHUNK = 4
_CCOLS = _CPW // _NCHUNK
_GPC = _CCOLS // 16
_CB = 20

_LN2 = 0.6931471805599453


_M_SCALE = _MAX_M * float(min(_CLS_NUM_LIST)) ** 0.25


def _poly_log(s):
    bits = plsc.bitcast(s, jnp.int32)
    e = ((bits >> 23) & 255) - 127
    mant = plsc.bitcast((bits & 0x7FFFFF) | 0x3F800000, jnp.float32)
    t = (mant - 1.0) / (mant + 1.0)
    t2 = t * t
    p = jnp.float32(1.0 / 9.0)
    for c in (1.0 / 7.0, 1.0 / 5.0, 1.0 / 3.0, 1.0):
        p = p * t2 + jnp.float32(c)
    return e.astype(jnp.float32) * jnp.float32(_LN2) + (2.0 * t) * p


def _make_sc_kernel():
    mesh = plsc.VectorSubcoreMesh(core_axis_name="c", subcore_axis_name="s")

    @functools.partial(
        pl.kernel,
        mesh=mesh,
        compiler_params=pltpu.CompilerParams(
            needs_layout_passes=False,
            disable_bounds_checks=True,
            skip_device_barrier=True,
        ),
        out_type=jax.ShapeDtypeStruct((_NW, 16), jnp.float32),
        scratch_types=[
            pltpu.VMEM((_C, _CPW), jnp.float32),
            pltpu.VMEM((_CPW,), jnp.int32),
            pltpu.VMEM((_CPW,), jnp.float32),
            pltpu.VMEM((128,), jnp.float32),
            pltpu.VMEM((16,), jnp.float32),
            pltpu.SemaphoreType.DMA,
            pltpu.SemaphoreType.DMA,
            pltpu.SemaphoreType.DMA,
            pltpu.SemaphoreType.DMA,
            pltpu.SemaphoreType.DMA,
            pltpu.SemaphoreType.DMA,
        ],
    )
    def k(x_hbm, t_hbm, w_hbm, out_hbm, x_v, t_v, w_v, m_v, acc_v,
          s0, s1, s2, s3, st, sw):
        wid = lax.axis_index("s") * 2 + lax.axis_index("c")
        col0 = wid * _CPW
        sems = [s0, s1, s2, s3]
        copies = [
            pltpu.async_copy(
                x_hbm.at[:, pl.ds(col0 + kk * _CCOLS, _CCOLS)],
                x_v.at[:, pl.ds(kk * _CCOLS, _CCOLS)],
                sems[kk],
            )
            for kk in range(_NCHUNK)
        ]
        t_copy = pltpu.async_copy(t_hbm.at[pl.ds(col0, _CPW)], t_v, st)
        w_copy = pltpu.async_copy(w_hbm.at[pl.ds(col0, _CPW)], w_v, sw)

        lane = lax.iota(jnp.int32, 16)
        for kk in range(8):
            idx1 = (lane + (kk * 16 + 1)).astype(jnp.float32)
            cls = (jnp.float32(5000.0) / idx1).astype(jnp.int32)
            lncls = _poly_log(cls.astype(jnp.float32))
            m_v[pl.ds(kk * 16, 16)] = jnp.float32(_M_SCALE) * jnp.exp(
                jnp.float32(-0.25) * lncls)

        t_copy.wait()
        w_copy.wait()
        ninf = jnp.full((16,), -3.0e38, jnp.float32)
        zero = jnp.zeros((16,), jnp.float32)

        def group(g, acc):
            cb = g * 16
            rows_b = cb + lane
            tvec = plsc.load_gather(t_v, [rows_b])
            wvec = plsc.load_gather(w_v, [rows_b])
            mt = plsc.load_gather(m_v, [tvec])
            xt = plsc.load_gather(x_v, [tvec, rows_b])
            xt_m = xt - mt
            plsc.store_scatter(x_v, [tvec, rows_b], xt_m)

            mx4 = [ninf, ninf, ninf, ninf]
            for j in range(_C):
                mx4[j % 4] = jnp.maximum(mx4[j % 4], x_v[j, pl.ds(cb, 16)])
            big_m = jnp.float32(_S) * jnp.maximum(
                jnp.maximum(mx4[0], mx4[1]), jnp.maximum(mx4[2], mx4[3]))

            zero_bf = jnp.zeros((32,), jnp.bfloat16)
            sm4 = [zero_bf, zero_bf, zero_bf, zero_bf]
            for j in range(0, _C, 2):
                a0 = jnp.float32(_S) * x_v[j, pl.ds(cb, 16)] - big_m
                a1 = jnp.float32(_S) * x_v[j + 1, pl.ds(cb, 16)] - big_m
                e = jnp.exp(plsc.pack(a0, a1, format=plsc.PackFormat.INTERLEAVED))
                sm4[(j // 2) % 4] = sm4[(j // 2) % 4] + e
            sbf = (sm4[0] + sm4[1]) + (sm4[2] + sm4[3])
            s0, s1 = plsc.unpack(sbf, format=plsc.PackFormat.INTERLEAVED)
            s = s0 + s1

            ce = _poly_log(s) + big_m - jnp.float32(_S) * xt_m
            return acc + ce * wvec

        acc = zero
        for kk in range(_NCHUNK):
            copies[kk].wait()
            acc = plsc.parallel_loop(
                kk * _GPC, (kk + 1) * _GPC, unroll=2, carry=acc)(group)

        acc_v[...] = acc
        pltpu.sync_copy(acc_v, out_hbm.at[wid])

    return k


def kernel(x, target, instance_weights):
    assert x.shape == (_B, _C) and x.dtype == jnp.float32
    partials = _make_sc_kernel()(
        x.T,
        target.astype(jnp.int32),
        instance_weights,
    )
    return jnp.sum(partials) * jnp.float32(1.0 / _B)


# speedup vs baseline: 1.0256x; 1.0256x over previous
import functools

import jax
import jax.numpy as jnp
from jax import lax
from jax.experimental import pallas as pl
from jax.experimental.pallas import tpu as pltpu
from jax.experimental.pallas import tpu_sc as plsc

_CLS_NUM_LIST = [5000 // (i + 1) for i in range(100)]
_MAX_M = 0.5
_S = 30.0

_B = 16384
_C = 100
_NW = 32
_CPW = _B // _NW
_NCHUNK = 2
_CCOLS = _CPW // _NCHUNK
_GPC = _CCOLS // 16
_CB = 20

_LN2 = 0.6931471805599453


_M_SCALE = _MAX_M * float(min(_CLS_NUM_LIST)) ** 0.25


def _poly_log(s):
    bits = plsc.bitcast(s, jnp.int32)
    e = ((bits >> 23) & 255) - 127
    mant = plsc.bitcast((bits & 0x7FFFFF) | 0x3F800000, jnp.float32)
    z = mant - 1.0
    p = jnp.float32(-0.01720778467569362)
    for c in (0.08172558065289895, -0.1887807207324388, 0.31458909833133447,
              -0.4969774040183165, 0.9997923579715677, 3.5112141751835285e-06):
        p = p * z + jnp.float32(c)
    return e.astype(jnp.float32) * jnp.float32(_LN2) + p


_MAG = 12582912.0
_MAGBITS = 0x4B400000
_LOG2E = 1.4426950408889634


def _exp_valu(a):
    ac = jnp.maximum(a, jnp.float32(-80.0))
    t = ac * jnp.float32(_LOG2E)
    tmp = t + jnp.float32(_MAG)
    n_f = tmp - jnp.float32(_MAG)
    f = t - n_f
    p = jnp.float32(0.055875501633782666)
    for c in (0.2422944444782086, 0.6931272662119475, 0.9999482435818646):
        p = p * f + jnp.float32(c)
    nbits = (plsc.bitcast(tmp, jnp.int32) - _MAGBITS) << 23
    return plsc.bitcast(plsc.bitcast(p, jnp.int32) + nbits, jnp.float32)


def _make_sc_kernel():
    mesh = plsc.VectorSubcoreMesh(core_axis_name="c", subcore_axis_name="s")

    @functools.partial(
        pl.kernel,
        mesh=mesh,
        compiler_params=pltpu.CompilerParams(
            needs_layout_passes=False,
            disable_bounds_checks=True,
            skip_device_barrier=True,
        ),
        out_type=jax.ShapeDtypeStruct((_NW, 16), jnp.float32),
        scratch_types=[
            pltpu.VMEM((_C, _CPW), jnp.float32),
            pltpu.VMEM((_CPW,), jnp.int32),
            pltpu.VMEM((_CPW,), jnp.float32),
            pltpu.VMEM((128,), jnp.float32),
            pltpu.VMEM((16,), jnp.float32),
            pltpu.SemaphoreType.DMA,
            pltpu.SemaphoreType.DMA,
            pltpu.SemaphoreType.DMA,
            pltpu.SemaphoreType.DMA,
            pltpu.SemaphoreType.DMA,
            pltpu.SemaphoreType.DMA,
        ],
    )
    def k(x_hbm, t_hbm, w_hbm, out_hbm, x_v, t_v, w_v, m_v, acc_v,
          s0, s1, s2, s3, st, sw):
        wid = lax.axis_index("s") * 2 + lax.axis_index("c")
        col0 = wid * _CPW
        sems = [s0, s1, s2, s3]
        copies = [
            pltpu.async_copy(
                x_hbm.at[:, pl.ds(col0 + kk * _CCOLS, _CCOLS)],
                x_v.at[:, pl.ds(kk * _CCOLS, _CCOLS)],
                sems[kk],
            )
            for kk in range(_NCHUNK)
        ]
        t_copy = pltpu.async_copy(t_hbm.at[pl.ds(col0, _CPW)], t_v, st)
        w_copy = pltpu.async_copy(w_hbm.at[pl.ds(col0, _CPW)], w_v, sw)

        lane = lax.iota(jnp.int32, 16)
        for kk in range(8):
            idx1 = (lane + (kk * 16 + 1)).astype(jnp.float32)
            cls = (jnp.float32(5000.0) / idx1).astype(jnp.int32)
            lncls = _poly_log(cls.astype(jnp.float32))
            m_v[pl.ds(kk * 16, 16)] = jnp.float32(_M_SCALE) * jnp.exp(
                jnp.float32(-0.25) * lncls)

        t_copy.wait()
        w_copy.wait()
        ninf = jnp.full((16,), -3.0e38, jnp.float32)
        zero = jnp.zeros((16,), jnp.float32)

        def group(g, acc):
            cb = g * 16
            rows_b = cb + lane
            tvec = plsc.load_gather(t_v, [rows_b])
            wvec = plsc.load_gather(w_v, [rows_b])
            mt = plsc.load_gather(m_v, [tvec])
            xt = plsc.load_gather(x_v, [tvec, rows_b])
            xt_m = xt - mt
            plsc.store_scatter(x_v, [tvec, rows_b], xt_m)

            mx4 = [ninf, ninf, ninf, ninf]
            for j in range(_C):
                mx4[j % 4] = jnp.maximum(mx4[j % 4], x_v[j, pl.ds(cb, 16)])
            big_m = jnp.float32(_S) * jnp.maximum(
                jnp.maximum(mx4[0], mx4[1]), jnp.maximum(mx4[2], mx4[3]))

            zero_bf = jnp.zeros((32,), jnp.bfloat16)
            smb = [zero_bf, zero_bf, zero_bf]
            smf = [zero, zero]
            for idx, j in enumerate(range(0, _C, 2)):
                a0 = jnp.float32(_S) * x_v[j, pl.ds(cb, 16)] - big_m
                a1 = jnp.float32(_S) * x_v[j + 1, pl.ds(cb, 16)] - big_m
                if idx % 3 == 2:
                    smf[0] = smf[0] + _exp_valu(a0)
                    smf[1] = smf[1] + _exp_valu(a1)
                else:
                    e = jnp.exp(
                        plsc.pack(a0, a1, format=plsc.PackFormat.INTERLEAVED))
                    smb[idx % 3] = smb[idx % 3] + e
            sbf = smb[0] + smb[1] + smb[2]
            s0, s1 = plsc.unpack(sbf, format=plsc.PackFormat.INTERLEAVED)
            s = (s0 + s1) + (smf[0] + smf[1])

            ce = _poly_log(s) + big_m - jnp.float32(_S) * xt_m
            return acc + ce * wvec

        acc = zero
        for kk in range(_NCHUNK):
            copies[kk].wait()
            acc = plsc.parallel_loop(
                kk * _GPC, (kk + 1) * _GPC, unroll=2, carry=acc)(group)

        acc_v[...] = acc
        pltpu.sync_copy(acc_v, out_hbm.at[wid])

    return k


def kernel(x, target, instance_weights):
    assert x.shape == (_B, _C) and x.dtype == jnp.float32
    partials = _make_sc_kernel()(
        x.T,
        target.astype(jnp.int32),
        instance_weights,
    )
    return jnp.sum(partials) * jnp.float32(1.0 / _B)


# speedup vs baseline: 1.0943x; 1.0670x over previous
import functools

import jax
import jax.numpy as jnp
from jax import lax
from jax.experimental import pallas as pl
from jax.experimental.pallas import tpu as pltpu
from jax.experimental.pallas import tpu_sc as plsc

_CLS_NUM_LIST = [5000 // (i + 1) for i in range(100)]
_MAX_M = 0.5
_S = 30.0

_B = 16384
_C = 100
_NW = 32
_CPW = _B // _NW
_NCHUNK = 2
_CCOLS = _CPW // _NCHUNK
_GPC = _CCOLS // 16
_CB = 20

_LN2 = 0.6931471805599453


_M_SCALE = _MAX_M * float(min(_CLS_NUM_LIST)) ** 0.25


def _poly_log(s):
    bits = plsc.bitcast(s, jnp.int32)
    e = ((bits >> 23) & 255) - 127
    mant = plsc.bitcast((bits & 0x7FFFFF) | 0x3F800000, jnp.float32)
    t = (mant - 1.0) / (mant + 1.0)
    t2 = t * t
    p = jnp.float32(1.0 / 9.0)
    for c in (1.0 / 7.0, 1.0 / 5.0, 1.0 / 3.0, 1.0):
        p = p * t2 + jnp.float32(c)
    return e.astype(jnp.float32) * jnp.float32(_LN2) + (2.0 * t) * p


def _make_sc_kernel():
    mesh = plsc.VectorSubcoreMesh(core_axis_name="c", subcore_axis_name="s")

    @functools.partial(
        pl.kernel,
        mesh=mesh,
        compiler_params=pltpu.CompilerParams(
            needs_layout_passes=False,
            disable_bounds_checks=True,
            skip_device_barrier=True,
        ),
        out_type=jax.ShapeDtypeStruct((_NW, 16), jnp.float32),
        scratch_types=[
            pltpu.VMEM((_C, _CPW), jnp.float32),
            pltpu.VMEM((_CPW,), jnp.int32),
            pltpu.VMEM((_CPW,), jnp.float32),
            pltpu.VMEM((128,), jnp.float32),
            pltpu.VMEM((16,), jnp.float32),
            pltpu.SemaphoreType.DMA,
            pltpu.SemaphoreType.DMA,
            pltpu.SemaphoreType.DMA,
            pltpu.SemaphoreType.DMA,
            pltpu.SemaphoreType.DMA,
            pltpu.SemaphoreType.DMA,
        ],
    )
    def k(x_hbm, t_hbm, w_hbm, out_hbm, x_v, t_v, w_v, m_v, acc_v,
          s0, s1, s2, s3, st, sw):
        wid = lax.axis_index("s") * 2 + lax.axis_index("c")
        col0 = wid * _CPW
        sems = [s0, s1, s2, s3]
        copies = [
            pltpu.async_copy(
                x_hbm.at[:, pl.ds(col0 + kk * _CCOLS, _CCOLS)],
                x_v.at[:, pl.ds(kk * _CCOLS, _CCOLS)],
                sems[kk],
            )
            for kk in range(_NCHUNK)
        ]
        t_copy = pltpu.async_copy(t_hbm.at[pl.ds(col0, _CPW)], t_v, st)
        w_copy = pltpu.async_copy(w_hbm.at[pl.ds(col0, _CPW)], w_v, sw)

        lane = lax.iota(jnp.int32, 16)
        for kk in range(8):
            idx1 = (lane + (kk * 16 + 1)).astype(jnp.float32)
            cls = (jnp.float32(5000.0) / idx1).astype(jnp.int32)
            lncls = _poly_log(cls.astype(jnp.float32))
            m_v[pl.ds(kk * 16, 16)] = jnp.float32(_M_SCALE) * jnp.exp(
                jnp.float32(-0.25) * lncls)

        t_copy.wait()
        w_copy.wait()
        ninf = jnp.full((16,), -3.0e38, jnp.float32)
        zero = jnp.zeros((16,), jnp.float32)

        def group(g, acc):
            cb = g * 16
            rows_b = cb + lane
            tvec = plsc.load_gather(t_v, [rows_b])
            wvec = plsc.load_gather(w_v, [rows_b])
            mt = plsc.load_gather(m_v, [tvec])
            xt = plsc.load_gather(x_v, [tvec, rows_b])
            xt_m = xt - mt
            plsc.store_scatter(x_v, [tvec, rows_b], xt_m)

            mx4 = [ninf, ninf, ninf, ninf]
            for j in range(_C):
                mx4[j % 4] = jnp.maximum(mx4[j % 4], x_v[j, pl.ds(cb, 16)])
            big_m = jnp.float32(_S) * jnp.maximum(
                jnp.maximum(mx4[0], mx4[1]), jnp.maximum(mx4[2], mx4[3]))

            zero_bf = jnp.zeros((32,), jnp.bfloat16)
            sm4 = [zero_bf, zero_bf, zero_bf, zero_bf]
            for j in range(0, _C, 2):
                a0 = jnp.float32(_S) * x_v[j, pl.ds(cb, 16)] - big_m
                a1 = jnp.float32(_S) * x_v[j + 1, pl.ds(cb, 16)] - big_m
                e = jnp.exp(plsc.pack(a0, a1, format=plsc.PackFormat.INTERLEAVED))
                sm4[(j // 2) % 4] = sm4[(j // 2) % 4] + e
            sbf = (sm4[0] + sm4[1]) + (sm4[2] + sm4[3])
            s0, s1 = plsc.unpack(sbf, format=plsc.PackFormat.INTERLEAVED)
            s = s0 + s1

            ce = _poly_log(s) + big_m - jnp.float32(_S) * xt_m
            return acc + ce * wvec

        acc = zero
        for kk in range(_NCHUNK):
            copies[kk].wait()
            acc = plsc.parallel_loop(
                kk * _GPC, (kk + 1) * _GPC, unroll=2, carry=acc)(group)

        acc_v[...] = acc
        pltpu.sync_copy(acc_v, out_hbm.at[wid])

    return k


def kernel(x, target, instance_weights):
    assert x.shape == (_B, _C) and x.dtype == jnp.float32
    partials = _make_sc_kernel()(
        x.T,
        target.astype(jnp.int32),
        instance_weights,
    )
    return jnp.sum(partials) * jnp.float32(1.0 / _B)
